# baseline (speedup 1.0000x reference)
import functools

import jax
import jax.numpy as jnp
from jax import lax
from jax.experimental import pallas as pl
from jax.experimental.pallas import tpu as pltpu
from jax.experimental.pallas import tpu_sc as plsc

D = 32
B = 16384
NC = 2
NS = 16
NW = NC * NS
BPW = B // NW
BATCH = 4
NBATCH = BPW // BATCH
L = 16


def _fire(ut3, vflat, uring, vstage, vidx, iidx_v, consts2,
          sem_u, sem_v, uvec, voff, slot_base, obase):
    opat, dbase = consts2
    for j in range(BATCH):
        u = uvec[voff + j]
        cu = jax.lax.shift_right_logical(u, 7)
        pltpu.async_copy(
            ut3.at[:, :, pl.ds(cu * 128, 128)],
            uring.at[slot_base + j], sem_u)
    itemv = plsc.load_gather(iidx_v, [obase + opat]) * D
    vbase = (slot_base // BATCH) * 128
    for t in range(8):
        vidx[pl.ds(vbase + t * L, L)] = itemv + (dbase + 4 * t)
    pltpu.async_copy(
        vflat.at[vidx.at[pl.ds(vbase, 128)]],
        vstage.at[pl.ds(vbase, 128)], sem_v)


def _drain(ut3, vflat, uring, vstage, sem_u, sem_v, slot_base):
    for j in range(BATCH):
        pltpu.make_async_copy(
            ut3.at[:, :, pl.ds(0, 128)], uring.at[slot_base + j], sem_u
        ).wait()
    vbase = (slot_base // BATCH) * 128
    pltpu.make_async_copy(
        vflat.at[pl.ds(0, 128)], vstage.at[pl.ds(vbase, 128)], sem_v
    ).wait()


def _process(uring, vstage, lanes_u, out_v, obase, slot_base, consts):
    iota, q4, fold_v = consts
    opos = obase + (iota & 3)
    lu = plsc.load_gather(lanes_u, [opos])
    slotv = slot_base + (iota & 3)
    vbase = (slot_base // BATCH) * 128
    acc = jnp.zeros((L,), jnp.float32)
    for t in range(8):
        cv = vstage[pl.ds(vbase + t * L, L)]
        d0 = 4 * t
        rv = jnp.full((L,), d0 // 8, jnp.int32)
        sv = (d0 % 8) + q4
        cu = plsc.load_gather(uring, [slotv, rv, sv, lu])
        acc = acc + cu * cv
    fold_v[...] = acc
    h = (plsc.load_gather(fold_v, [iota & 7])
         + plsc.load_gather(fold_v, [(iota & 7) + 8]))
    fold_v[...] = h
    out4 = (plsc.load_gather(fold_v, [iota & 3])
            + plsc.load_gather(fold_v, [(iota & 3) + 4]))
    plsc.store_scatter(out_v, [opos], out4, mask=iota < BATCH)


def _body(ut3, vflat, uidx_hbm, iidx_hbm, out_hbm,
          uring, vstage, vidx, uidx_v, iidx_v, lanes_u, out_v, fold_v,
          sem_u0, sem_v0, sem_u1, sem_v1, sem_u2, sem_v2, sem_u3, sem_v3):
    wid = lax.axis_index("s") * NC + lax.axis_index("c")
    base = wid * BPW

    pltpu.sync_copy(uidx_hbm.at[pl.ds(base, BPW)], uidx_v)
    pltpu.sync_copy(iidx_hbm.at[pl.ds(base, BPW)], iidx_v)

    iota = lax.iota(jnp.int32, L)
    q4 = jax.lax.shift_right_logical(iota, 2)
    opat = iota & 3
    dbase = jax.lax.shift_right_logical(iota, 2)
    consts = (iota, q4, fold_v)
    consts2 = (opat, dbase)
    for k in range(BPW // L):
        lanes_u[pl.ds(k * L, L)] = uidx_v[pl.ds(k * L, L)] & 127

    fire = functools.partial(
        _fire, ut3, vflat, uring, vstage, vidx, iidx_v, consts2)
    drain = functools.partial(_drain, ut3, vflat, uring, vstage)
    proc = functools.partial(
        _process, uring, vstage, lanes_u, out_v, consts=consts)

    sems = ((sem_u0, sem_v0), (sem_u1, sem_v1),
            (sem_u2, sem_v2), (sem_u3, sem_v3))
    pvec_u = uidx_v[pl.ds(0, L)]
    fire(*sems[0], pvec_u, 0, 0, 0)
    fire(*sems[1], pvec_u, 4, BATCH, 4)

    def step(i, carry):
        base16 = i * L
        uvec = uidx_v[pl.ds(base16, L)]
        nbase = jnp.minimum(base16 + L, BPW - L)
        nuvec = uidx_v[pl.ds(nbase, L)]
        for b in range(4):
            fp = (b + 2) % 4
            if b < 2:
                fire(*sems[fp], uvec, 4 * b + 8, fp * BATCH,
                     base16 + 4 * b + 8)
            else:
                @pl.when(i < NBATCH // 4 - 1)
                def _():
                    fire(*sems[fp], nuvec, 4 * b - 8, fp * BATCH,
                         base16 + 4 * b + 8)
            drain(*sems[b], b * BATCH)
            proc(base16 + 4 * b, b * BATCH)
        return carry

    lax.fori_loop(0, NBATCH // 4, step, 0)

    pltpu.sync_copy(out_v, out_hbm.at[pl.ds(base, BPW)])


@jax.jit
def _run(Ut3, Vflat, user_idx, item_idx):
    mesh = plsc.VectorSubcoreMesh(core_axis_name="c", subcore_axis_name="s")
    f = functools.partial(
        pl.kernel,
        out_type=jax.ShapeDtypeStruct((B,), jnp.float32),
        mesh=mesh,
        compiler_params=pltpu.CompilerParams(
            use_tc_tiling_on_sc=True,
            needs_layout_passes=False,
        ),
        scratch_types=[
            pltpu.VMEM((4 * BATCH, 4, 8, 128), jnp.float32),
            pltpu.VMEM((512,), jnp.float32),
            pltpu.VMEM((512,), jnp.int32),
            pltpu.VMEM((BPW,), jnp.int32),
            pltpu.VMEM((BPW,), jnp.int32),
            pltpu.VMEM((BPW,), jnp.int32),
            pltpu.VMEM((BPW,), jnp.float32),
            pltpu.VMEM((L,), jnp.float32),
            pltpu.SemaphoreType.DMA,
            pltpu.SemaphoreType.DMA,
            pltpu.SemaphoreType.DMA,
            pltpu.SemaphoreType.DMA,
            pltpu.SemaphoreType.DMA,
            pltpu.SemaphoreType.DMA,
            pltpu.SemaphoreType.DMA,
            pltpu.SemaphoreType.DMA,
        ],
    )(_body)
    return f(Ut3, Vflat, user_idx, item_idx)


def kernel(U, V, user_idx, item_idx):
    Ut3 = U.T.reshape(4, 8, U.shape[0])
    Vflat = V.reshape(-1)
    return _run(Ut3, Vflat,
                user_idx.astype(jnp.int32), item_idx.astype(jnp.int32))


# speedup vs baseline: 1.0447x; 1.0447x over previous
import functools

import jax
import jax.numpy as jnp
from jax import lax
from jax.experimental import pallas as pl
from jax.experimental.pallas import tpu as pltpu
from jax.experimental.pallas import tpu_sc as plsc

D = 32
B = 16384
NC = 2
NS = 16
NW = NC * NS
BPW = B // NW
BATCH = 4
NBATCH = BPW // BATCH
L = 16


def _fire(ut3, vflat, uring, vstage, vidx, iidx_v, consts2,
          sem_u, sem_v, uvec, voff, slot_base, obase):
    opat, dbase = consts2
    for j in range(BATCH):
        u = uvec[voff + j]
        cu = jax.lax.shift_right_logical(u, 7)
        pltpu.async_copy(
            ut3.at[:, :, pl.ds(cu * 128, 128)],
            uring.at[slot_base + j], sem_u)
    itemv = plsc.load_gather(iidx_v, [obase + opat]) * D
    vbase = (slot_base // BATCH) * 128
    for t in range(8):
        vidx[pl.ds(vbase + t * L, L)] = itemv + (dbase + 4 * t)
    pltpu.async_copy(
        vflat.at[vidx.at[pl.ds(vbase, 128)]],
        vstage.at[pl.ds(vbase, 128)], sem_v)


def _drain(ut3, vflat, uring, vstage, sem_u, sem_v, slot_base):
    for j in range(BATCH):
        pltpu.make_async_copy(
            ut3.at[:, :, pl.ds(0, 128)], uring.at[slot_base + j], sem_u
        ).wait()
    vbase = (slot_base // BATCH) * 128
    pltpu.make_async_copy(
        vflat.at[pl.ds(0, 128)], vstage.at[pl.ds(vbase, 128)], sem_v
    ).wait()


def _process(uring, vstage, lanes_u, out_v, obase, slot_base, consts):
    iota, q4, fold_v = consts
    opos = obase + (iota & 3)
    lu = plsc.load_gather(lanes_u, [opos])
    slotv = slot_base + (iota & 3)
    vbase = (slot_base // BATCH) * 128
    acc = jnp.zeros((L,), jnp.float32)
    for t in range(8):
        cv = vstage[pl.ds(vbase + t * L, L)]
        d0 = 4 * t
        rv = jnp.full((L,), d0 // 8, jnp.int32)
        sv = (d0 % 8) + q4
        cu = plsc.load_gather(uring, [slotv, rv, sv, lu])
        acc = acc + cu * cv
    fold_v[...] = acc
    h = (plsc.load_gather(fold_v, [iota & 7])
         + plsc.load_gather(fold_v, [(iota & 7) + 8]))
    fold_v[...] = h
    out4 = (plsc.load_gather(fold_v, [iota & 3])
            + plsc.load_gather(fold_v, [(iota & 3) + 4]))
    plsc.store_scatter(out_v, [opos], out4, mask=iota < BATCH)


def _body(ut3, vflat, uidx_hbm, iidx_hbm, out_hbm,
          uring, vstage, vidx, uidx_v, iidx_v, lanes_u, out_v, fold_v,
          sem_u0, sem_v0, sem_u1, sem_v1, sem_u2, sem_v2, sem_u3, sem_v3):
    wid = lax.axis_index("s") * NC + lax.axis_index("c")
    base = wid * BPW

    pltpu.sync_copy(uidx_hbm.at[pl.ds(base, BPW)], uidx_v)
    pltpu.sync_copy(iidx_hbm.at[pl.ds(base, BPW)], iidx_v)

    iota = lax.iota(jnp.int32, L)
    q4 = jax.lax.shift_right_logical(iota, 2)
    opat = iota & 3
    dbase = jax.lax.shift_right_logical(iota, 2)
    consts = (iota, q4, fold_v)
    consts2 = (opat, dbase)
    for k in range(BPW // L):
        lanes_u[pl.ds(k * L, L)] = uidx_v[pl.ds(k * L, L)] & 127

    fire = functools.partial(
        _fire, ut3, vflat, uring, vstage, vidx, iidx_v, consts2)
    drain = functools.partial(_drain, ut3, vflat, uring, vstage)
    proc = functools.partial(
        _process, uring, vstage, lanes_u, out_v, consts=consts)

    sems = ((sem_u0, sem_v0), (sem_u1, sem_v1),
            (sem_u2, sem_v2), (sem_u3, sem_v3))
    pvec_u = uidx_v[pl.ds(0, L)]
    fire(*sems[0], pvec_u, 0, 0, 0)
    fire(*sems[1], pvec_u, 4, BATCH, 4)
    fire(*sems[2], pvec_u, 8, 2 * BATCH, 8)

    def step(i, carry):
        base16 = i * L
        uvec = uidx_v[pl.ds(base16, L)]
        nbase = jnp.minimum(base16 + L, BPW - L)
        nuvec = uidx_v[pl.ds(nbase, L)]
        for b in range(4):
            fp = (b + 3) % 4
            if b < 1:
                fire(*sems[fp], uvec, 4 * b + 12, fp * BATCH,
                     base16 + 4 * b + 12)
            else:
                @pl.when(i < NBATCH // 4 - 1)
                def _():
                    fire(*sems[fp], nuvec, 4 * b - 4, fp * BATCH,
                         base16 + 4 * b + 12)
            drain(*sems[b], b * BATCH)
            proc(base16 + 4 * b, b * BATCH)
        return carry

    lax.fori_loop(0, NBATCH // 4, step, 0)

    pltpu.sync_copy(out_v, out_hbm.at[pl.ds(base, BPW)])


@jax.jit
def _run(Ut3, Vflat, user_idx, item_idx):
    mesh = plsc.VectorSubcoreMesh(core_axis_name="c", subcore_axis_name="s")
    f = functools.partial(
        pl.kernel,
        out_type=jax.ShapeDtypeStruct((B,), jnp.float32),
        mesh=mesh,
        compiler_params=pltpu.CompilerParams(
            use_tc_tiling_on_sc=True,
            needs_layout_passes=False,
        ),
        scratch_types=[
            pltpu.VMEM((4 * BATCH, 4, 8, 128), jnp.float32),
            pltpu.VMEM((512,), jnp.float32),
            pltpu.VMEM((512,), jnp.int32),
            pltpu.VMEM((BPW,), jnp.int32),
            pltpu.VMEM((BPW,), jnp.int32),
            pltpu.VMEM((BPW,), jnp.int32),
            pltpu.VMEM((BPW,), jnp.float32),
            pltpu.VMEM((L,), jnp.float32),
            pltpu.SemaphoreType.DMA,
            pltpu.SemaphoreType.DMA,
            pltpu.SemaphoreType.DMA,
            pltpu.SemaphoreType.DMA,
            pltpu.SemaphoreType.DMA,
            pltpu.SemaphoreType.DMA,
            pltpu.SemaphoreType.DMA,
            pltpu.SemaphoreType.DMA,
        ],
    )(_body)
    return f(Ut3, Vflat, user_idx, item_idx)


def kernel(U, V, user_idx, item_idx):
    Ut3 = U.T.reshape(4, 8, U.shape[0])
    Vflat = V.reshape(-1)
    return _run(Ut3, Vflat,
                user_idx.astype(jnp.int32), item_idx.astype(jnp.int32))
